# baseline (speedup 1.0000x reference)
import functools

import jax
import jax.numpy as jnp
from jax import lax
from jax.experimental import pallas as pl
from jax.experimental.pallas import tpu as pltpu
from jax.experimental.pallas import tpu_sc as plsc

N = 10000
NP = 10240
D = 128
PD = 16
H = 128
E = 320000
G = 64
NCLS = 10

NC = 2
NS = 16
NW = NC * NS
EPW = 10240
EP = EPW * NW
ECH = EPW // 128
NI = 4
NBUF = 2
NGRP = ECH // NI
RPS = NP // NS
RPW = NP // NW
RB = 128
NBLK = NP // RB

f32 = jnp.float32
i32 = jnp.int32

def _sc_mesh():
    return plsc.VectorSubcoreMesh(
        core_axis_name="c", subcore_axis_name="s", num_cores=NC, num_subcores=NS)


@functools.cache
def _get_embed_sc():
    @functools.partial(
        pl.kernel,
        out_type=jax.ShapeDtypeStruct((NP, PD), f32),
        scratch_types=[
            pltpu.VMEM((80,), i32),
            pltpu.VMEM((80, PD), f32),
            pltpu.SemaphoreType.DMA,
        ],
        mesh=_sc_mesh(),
        compiler_params=pltpu.CompilerParams(use_tc_tiling_on_sc=False),
    )
    def _embed_sc(ports_hbm, ptab_hbm, pe_hbm, pidx, perows, sem):
        c = lax.axis_index("c")
        s = lax.axis_index("s")
        wid = s * NC + c
        base = wid * RPW

        @pl.loop(0, RPW // 80)
        def _chunk(j):
            off = base + j * 80
            pltpu.sync_copy(ports_hbm.at[pl.ds(off, 80)], pidx)
            pltpu.async_copy(ptab_hbm.at[pidx], perows, sem).wait()
            pltpu.sync_copy(perows, pe_hbm.at[pl.ds(off, 80)])

    return _embed_sc


@functools.cache
def _get_edge_sc():
    @functools.partial(
        pl.kernel,
        out_type=(
            jax.ShapeDtypeStruct((NP, H), f32),
            jax.ShapeDtypeStruct((NP, H), f32),
        ),
        scratch_types=[
            pltpu.VMEM((NI, 2, 128), i32),
            pltpu.VMEM((NBUF, 128, H), f32),
            pltpu.VMEM_SHARED((NP, H), f32),
            pltpu.SemaphoreType.DMA((NI,)),
            pltpu.SemaphoreType.DMA((NBUF,)),
            pltpu.SemaphoreType.DMA((NBUF,)),
        ],
        mesh=_sc_mesh(),
    )
    def _edge_sc(hw_hbm, epk_hbm, zeros_hbm, a0_hbm, a1_hbm,
                 eidx, rows, acc, isem, gsem, ssem):
        c = lax.axis_index("c")
        s = lax.axis_index("s")
        pltpu.sync_copy(zeros_hbm.at[pl.ds(s * RPS, RPS)],
                        acc.at[pl.ds(s * RPS, RPS)])
        plsc.subcore_barrier()
        ncH = ECH * NC
        ngrp = ncH // NI
        qb = s * ncH

        def idx_d(k, j):
            return pltpu.make_async_copy(epk_hbm.at[qb + k], eidx.at[j],
                                         isem.at[j])

        def gat_d(b, j):
            return pltpu.make_async_copy(hw_hbm.at[eidx.at[j, 0]], rows.at[b],
                                         gsem.at[b])

        def sct_d(b, j):
            return pltpu.make_async_copy(rows.at[b], acc.at[eidx.at[j, 1]],
                                         ssem.at[b])

        @pl.when(c == 0)
        def _pipeline():
            for j in range(NBUF):
                idx_d(j, j).start()

            @pl.loop(0, ngrp)
            def _grp(g):
                for u in range(NI):
                    b = u % NBUF
                    k = g * NI + u
                    if u < NBUF:
                        @pl.when(g > 0)
                        def _():
                            sct_d(b, u + NBUF).wait()
                    else:
                        sct_d(b, u - NBUF).wait()
                    idx_d(k, u).wait()
                    gat_d(b, u).start()
                    j2 = (u + NBUF) % NI
                    if u < NBUF:
                        idx_d(k + NBUF, j2).start()
                    else:
                        @pl.when(g < ngrp - 1)
                        def _():
                            idx_d(k + NBUF, j2).start()
                    if u >= 1:
                        bp = (u - 1) % NBUF
                        gat_d(bp, u - 1).wait()
                        sct_d(bp, u - 1).start(add=True)
                    else:
                        @pl.when(g > 0)
                        def _():
                            gat_d((NI - 1) % NBUF, NI - 1).wait()
                            sct_d((NI - 1) % NBUF, NI - 1).start(add=True)

            gat_d((NI - 1) % NBUF, NI - 1).wait()
            sct_d((NI - 1) % NBUF, NI - 1).start(add=True)
            for u in range(NI - NBUF, NI):
                sct_d(u % NBUF, u).wait()

        plsc.subcore_barrier()

        @pl.when(c == 0)
        def _():
            pltpu.sync_copy(acc.at[pl.ds(s * RPS, RPS)],
                            a0_hbm.at[pl.ds(s * RPS, RPS)])

        @pl.when(c == 1)
        def _():
            pltpu.sync_copy(acc.at[pl.ds(s * RPS, RPS)],
                            a1_hbm.at[pl.ds(s * RPS, RPS)])

    return _edge_sc


def _dot(a, b):
    return jnp.dot(a, b, preferred_element_type=f32)


def _dense0_body(x_ref, pe_ref, fl_ref, wxr, wpr, twr, wxt, wpt, twt, br,
                 hw_ref, hr_ref):
    x = x_ref[...]
    pe = pe_ref[...]
    oh = (fl_ref[...] == lax.broadcasted_iota(i32, (1, 256), 1)).astype(f32)
    hw_ref[...] = _dot(x, wxr[...]) + _dot(pe, wpr[...]) + _dot(oh, twr[...])
    hr_ref[...] = (_dot(x, wxt[...]) + _dot(pe, wpt[...]) + _dot(oh, twt[...])
                   + br[...])


_dense0 = pl.pallas_call(
    _dense0_body,
    grid=(NBLK,),
    in_specs=[
        pl.BlockSpec((RB, D), lambda i: (i, 0)),
        pl.BlockSpec((RB, PD), lambda i: (i, 0)),
        pl.BlockSpec((RB, 1), lambda i: (i, 0)),
        pl.BlockSpec((D, H), lambda i: (0, 0)),
        pl.BlockSpec((PD, H), lambda i: (0, 0)),
        pl.BlockSpec((256, H), lambda i: (0, 0)),
        pl.BlockSpec((D, H), lambda i: (0, 0)),
        pl.BlockSpec((PD, H), lambda i: (0, 0)),
        pl.BlockSpec((256, H), lambda i: (0, 0)),
        pl.BlockSpec((1, H), lambda i: (0, 0)),
    ],
    out_specs=(
        pl.BlockSpec((RB, H), lambda i: (i, 0)),
        pl.BlockSpec((RB, H), lambda i: (i, 0)),
    ),
    out_shape=(
        jax.ShapeDtypeStruct((NP, H), f32),
        jax.ShapeDtypeStruct((NP, H), f32),
    ),
)


def _densemid_body(a0_ref, a1_ref, hrin_ref, wr, wt, b, hw_ref, hr_ref):
    h = jnp.maximum(a0_ref[...] + a1_ref[...] + hrin_ref[...], 0.0)
    hw_ref[...] = _dot(h, wr[...])
    hr_ref[...] = _dot(h, wt[...]) + b[...]


_densemid = pl.pallas_call(
    _densemid_body,
    grid=(NBLK,),
    in_specs=[
        pl.BlockSpec((RB, H), lambda i: (i, 0)),
        pl.BlockSpec((RB, H), lambda i: (i, 0)),
        pl.BlockSpec((RB, H), lambda i: (i, 0)),
        pl.BlockSpec((H, H), lambda i: (0, 0)),
        pl.BlockSpec((H, H), lambda i: (0, 0)),
        pl.BlockSpec((1, H), lambda i: (0, 0)),
    ],
    out_specs=(
        pl.BlockSpec((RB, H), lambda i: (i, 0)),
        pl.BlockSpec((RB, H), lambda i: (i, 0)),
    ),
    out_shape=(
        jax.ShapeDtypeStruct((NP, H), f32),
        jax.ShapeDtypeStruct((NP, H), f32),
    ),
)


def _final_body(a0_ref, a1_ref, hrin_ref, b_ref, w0, b0, w1, b1, w2, b2,
                out_ref, pooled):
    i = pl.program_id(0)

    @pl.when(i == 0)
    def _():
        pooled[...] = jnp.full((G, H), -jnp.inf, f32)

    h = jnp.maximum(a0_ref[...] + a1_ref[...] + hrin_ref[...], 0.0)
    bb = b_ref[...]
    gmin = jnp.min(bb)
    gmax = jnp.minimum(jnp.max(bb), G - 1)

    def upd(g, carry):
        m = bb == g
        v = jnp.max(jnp.where(m, h, -jnp.inf), axis=0, keepdims=True)
        pooled[pl.ds(g, 1), :] = jnp.maximum(pooled[pl.ds(g, 1), :], v)
        return carry

    lax.fori_loop(gmin, gmax + 1, upd, 0)

    @pl.when(i == NBLK - 1)
    def _():
        p = pooled[...]
        p1 = jnp.maximum(_dot(p, w0[...]) + b0[...], 0.0)
        p2 = jnp.maximum(_dot(p1, w1[...]) + b1[...], 0.0)
        out_ref[...] = _dot(p2, w2[...]) + b2[...]


_final = pl.pallas_call(
    _final_body,
    grid=(NBLK,),
    in_specs=[
        pl.BlockSpec((RB, H), lambda i: (i, 0)),
        pl.BlockSpec((RB, H), lambda i: (i, 0)),
        pl.BlockSpec((RB, H), lambda i: (i, 0)),
        pl.BlockSpec((RB, 1), lambda i: (i, 0)),
        pl.BlockSpec((H, H), lambda i: (0, 0)),
        pl.BlockSpec((1, H), lambda i: (0, 0)),
        pl.BlockSpec((H, H // 2), lambda i: (0, 0)),
        pl.BlockSpec((1, H // 2), lambda i: (0, 0)),
        pl.BlockSpec((H // 2, H), lambda i: (0, 0)),
        pl.BlockSpec((1, H), lambda i: (0, 0)),
    ],
    out_specs=pl.BlockSpec((G, H), lambda i: (0, 0)),
    out_shape=jax.ShapeDtypeStruct((G, H), f32),
    scratch_shapes=[pltpu.VMEM((G, H), f32)],
)


def kernel(x, dst_ports, tcp_flags, edge_index, batch, port_table, tcp_table,
           W_rel0, W_root0, b0, W_rel1, W_root1, b1, W_rel2, W_root2, b2,
           fc_W0, fc_b0, fc_W1, fc_b1, fc_W2, fc_b2):
    x_p = jnp.concatenate([x.astype(f32), jnp.zeros((NP - N, D), f32)], axis=0)
    ports_p = jnp.concatenate(
        [dst_ports.astype(i32), jnp.zeros((NP - N,), i32)], axis=0)
    flags_p = jnp.concatenate(
        [tcp_flags.astype(i32), jnp.zeros((NP - N,), i32)], axis=0
    ).reshape(NP, 1)
    src_p = jnp.concatenate(
        [edge_index[0].astype(i32), jnp.zeros((EP - E,), i32)], axis=0)
    dst_p = jnp.concatenate(
        [edge_index[1].astype(i32), jnp.full((EP - E,), NP - 1, i32)], axis=0)
    epk = jnp.stack([src_p, dst_p], axis=0).reshape(2, NW * ECH, 128)
    epk = epk.transpose(1, 0, 2)
    batch_p = jnp.concatenate(
        [batch.astype(i32), jnp.full((NP - N,), G, i32)], axis=0
    ).reshape(NP, 1)
    zeros = jnp.zeros((NP, H), f32)

    Wxr0, Wpr0, Wfr0 = W_rel0[:D], W_rel0[D:D + PD], W_rel0[D + PD:]
    Wxt0, Wpt0, Wft0 = W_root0[:D], W_root0[D:D + PD], W_root0[D + PD:]
    TWr = tcp_table.astype(f32) @ Wfr0
    TWt = tcp_table.astype(f32) @ Wft0
    b0r = b0.reshape(1, H)
    b1r = b1.reshape(1, H)
    b2r = b2.reshape(1, H)
    fb0 = fc_b0.reshape(1, H)
    fb1 = fc_b1.reshape(1, H // 2)
    fW2p = jnp.concatenate([fc_W2, jnp.zeros((H // 2, H - NCLS), f32)], axis=1)
    fb2p = jnp.concatenate([fc_b2, jnp.zeros((H - NCLS,), f32)]).reshape(1, H)

    embed_sc = _get_embed_sc()
    edge_sc = _get_edge_sc()
    pe = embed_sc(ports_p, port_table.astype(f32))
    hw, hr = _dense0(x_p, pe, flags_p, Wxr0, Wpr0, TWr, Wxt0, Wpt0, TWt, b0r)
    a0, a1 = edge_sc(hw, epk, zeros)
    hw, hr = _densemid(a0, a1, hr, W_rel1, W_root1, b1r)
    a0, a1 = edge_sc(hw, epk, zeros)
    hw, hr = _densemid(a0, a1, hr, W_rel2, W_root2, b2r)
    a0, a1 = edge_sc(hw, epk, zeros)
    out = _final(a0, a1, hr, batch_p, fc_W0, fb0, fc_W1, fb1, fW2p, fb2p)
    return out[:, :NCLS]


# speedup vs baseline: 2.3574x; 2.3574x over previous
import functools

import jax
import jax.numpy as jnp
from jax import lax
from jax.experimental import pallas as pl
from jax.experimental.pallas import tpu as pltpu
from jax.experimental.pallas import tpu_sc as plsc

N = 10000
NP = 10240
D = 128
PD = 16
H = 128
E = 320000
G = 64
NCLS = 10

NC = 2
NS = 16
NW = NC * NS
EPW = 10240
EP = EPW * NW
ECH = EPW // 128
NI = 4
NBUF = 2
HH = H // 2
NCH = ECH * NC
NGRP = NCH // NI
RPS = NP // NS
RPW = NP // NW
RB = 128
NBLK = NP // RB

f32 = jnp.float32
i32 = jnp.int32

def _sc_mesh():
    return plsc.VectorSubcoreMesh(
        core_axis_name="c", subcore_axis_name="s", num_cores=NC, num_subcores=NS)


@functools.cache
def _get_embed_sc():
    @functools.partial(
        pl.kernel,
        out_type=jax.ShapeDtypeStruct((NP, PD), f32),
        scratch_types=[
            pltpu.VMEM((80,), i32),
            pltpu.VMEM((80, PD), f32),
            pltpu.SemaphoreType.DMA,
        ],
        mesh=_sc_mesh(),
        compiler_params=pltpu.CompilerParams(use_tc_tiling_on_sc=False),
    )
    def _embed_sc(ports_hbm, ptab_hbm, pe_hbm, pidx, perows, sem):
        c = lax.axis_index("c")
        s = lax.axis_index("s")
        wid = s * NC + c
        base = wid * RPW

        @pl.loop(0, RPW // 80)
        def _chunk(j):
            off = base + j * 80
            pltpu.sync_copy(ports_hbm.at[pl.ds(off, 80)], pidx)
            pltpu.async_copy(ptab_hbm.at[pidx], perows, sem).wait()
            pltpu.sync_copy(perows, pe_hbm.at[pl.ds(off, 80)])

    return _embed_sc


@functools.cache
def _get_edge_sc():
    @functools.partial(
        pl.kernel,
        out_type=(
            jax.ShapeDtypeStruct((NP, HH), f32),
            jax.ShapeDtypeStruct((NP, HH), f32),
        ),
        scratch_types=[
            pltpu.VMEM((NI, 2, 128), i32),
            pltpu.VMEM((NBUF, 128, HH), f32),
            pltpu.VMEM_SHARED((NP, HH), f32),
            pltpu.VMEM_SHARED((NP, HH), f32),
            pltpu.SemaphoreType.DMA((NI,)),
            pltpu.SemaphoreType.DMA((NBUF,)),
            pltpu.SemaphoreType.DMA((NBUF,)),
        ],
        mesh=_sc_mesh(),
        compiler_params=pltpu.CompilerParams(use_tc_tiling_on_sc=False),
    )
    def _edge_sc(hwa_hbm, hwb_hbm, epk_hbm, zeros_hbm, aa_hbm, ab_hbm,
                 eidx, rows, hws, acc, isem, gsem, ssem):
        c = lax.axis_index("c")
        s = lax.axis_index("s")
        pltpu.sync_copy(zeros_hbm.at[pl.ds(s * RPS, RPS)],
                        acc.at[pl.ds(s * RPS, RPS)])

        @pl.when(c == 0)
        def _():
            pltpu.sync_copy(hwa_hbm.at[pl.ds(s * RPS, RPS)],
                            hws.at[pl.ds(s * RPS, RPS)])

        @pl.when(c == 1)
        def _():
            pltpu.sync_copy(hwb_hbm.at[pl.ds(s * RPS, RPS)],
                            hws.at[pl.ds(s * RPS, RPS)])

        plsc.subcore_barrier()
        qb = s * NCH

        def idx_d(k, j):
            return pltpu.make_async_copy(epk_hbm.at[qb + k], eidx.at[j],
                                         isem.at[j])

        def gat_d(b, j):
            return pltpu.make_async_copy(hws.at[eidx.at[j, 0]], rows.at[b],
                                         gsem.at[b])

        def sct_d(b, j):
            return pltpu.make_async_copy(rows.at[b], acc.at[eidx.at[j, 1]],
                                         ssem.at[b])

        for j in range(NBUF):
            idx_d(j, j).start()

        @pl.loop(0, NGRP)
        def _grp(g):
            for u in range(NI):
                b = u % NBUF
                k = g * NI + u
                if u < NBUF:
                    @pl.when(g > 0)
                    def _():
                        sct_d(b, u + NBUF).wait()
                else:
                    sct_d(b, u - NBUF).wait()
                idx_d(k, u).wait()
                gat_d(b, u).start()
                j2 = (u + NBUF) % NI
                if u < NBUF:
                    idx_d(k + NBUF, j2).start()
                else:
                    @pl.when(g < NGRP - 1)
                    def _():
                        idx_d(k + NBUF, j2).start()
                if u >= 1:
                    bp = (u - 1) % NBUF
                    gat_d(bp, u - 1).wait()
                    sct_d(bp, u - 1).start(add=True)
                else:
                    @pl.when(g > 0)
                    def _():
                        gat_d((NI - 1) % NBUF, NI - 1).wait()
                        sct_d((NI - 1) % NBUF, NI - 1).start(add=True)

        gat_d((NI - 1) % NBUF, NI - 1).wait()
        sct_d((NI - 1) % NBUF, NI - 1).start(add=True)
        for u in range(NI - NBUF, NI):
            sct_d(u % NBUF, u).wait()

        plsc.subcore_barrier()

        @pl.when(c == 0)
        def _():
            pltpu.sync_copy(acc.at[pl.ds(s * RPS, RPS)],
                            aa_hbm.at[pl.ds(s * RPS, RPS)])

        @pl.when(c == 1)
        def _():
            pltpu.sync_copy(acc.at[pl.ds(s * RPS, RPS)],
                            ab_hbm.at[pl.ds(s * RPS, RPS)])

    return _edge_sc


def _dot(a, b):
    return jnp.dot(a, b, preferred_element_type=f32)


def _dense0_body(x_ref, pe_ref, fl_ref, wxr, wpr, twr, wxt, wpt, twt, br,
                 hwa_ref, hwb_ref, hr_ref):
    x = x_ref[...]
    pe = pe_ref[...]
    oh = (fl_ref[...] == lax.broadcasted_iota(i32, (1, 256), 1)).astype(f32)
    hw = _dot(x, wxr[...]) + _dot(pe, wpr[...]) + _dot(oh, twr[...])
    hwa_ref[...] = hw[:, :HH]
    hwb_ref[...] = hw[:, HH:]
    hr_ref[...] = (_dot(x, wxt[...]) + _dot(pe, wpt[...]) + _dot(oh, twt[...])
                   + br[...])


_dense0 = pl.pallas_call(
    _dense0_body,
    grid=(NBLK,),
    in_specs=[
        pl.BlockSpec((RB, D), lambda i: (i, 0)),
        pl.BlockSpec((RB, PD), lambda i: (i, 0)),
        pl.BlockSpec((RB, 1), lambda i: (i, 0)),
        pl.BlockSpec((D, H), lambda i: (0, 0)),
        pl.BlockSpec((PD, H), lambda i: (0, 0)),
        pl.BlockSpec((256, H), lambda i: (0, 0)),
        pl.BlockSpec((D, H), lambda i: (0, 0)),
        pl.BlockSpec((PD, H), lambda i: (0, 0)),
        pl.BlockSpec((256, H), lambda i: (0, 0)),
        pl.BlockSpec((1, H), lambda i: (0, 0)),
    ],
    out_specs=(
        pl.BlockSpec((RB, HH), lambda i: (i, 0)),
        pl.BlockSpec((RB, HH), lambda i: (i, 0)),
        pl.BlockSpec((RB, H), lambda i: (i, 0)),
    ),
    out_shape=(
        jax.ShapeDtypeStruct((NP, HH), f32),
        jax.ShapeDtypeStruct((NP, HH), f32),
        jax.ShapeDtypeStruct((NP, H), f32),
    ),
)


def _densemid_body(aa_ref, ab_ref, hrin_ref, wr, wt, b,
                   hwa_ref, hwb_ref, hr_ref):
    a = jnp.concatenate([aa_ref[...], ab_ref[...]], axis=1)
    h = jnp.maximum(a + hrin_ref[...], 0.0)
    hw = _dot(h, wr[...])
    hwa_ref[...] = hw[:, :HH]
    hwb_ref[...] = hw[:, HH:]
    hr_ref[...] = _dot(h, wt[...]) + b[...]


_densemid = pl.pallas_call(
    _densemid_body,
    grid=(NBLK,),
    in_specs=[
        pl.BlockSpec((RB, HH), lambda i: (i, 0)),
        pl.BlockSpec((RB, HH), lambda i: (i, 0)),
        pl.BlockSpec((RB, H), lambda i: (i, 0)),
        pl.BlockSpec((H, H), lambda i: (0, 0)),
        pl.BlockSpec((H, H), lambda i: (0, 0)),
        pl.BlockSpec((1, H), lambda i: (0, 0)),
    ],
    out_specs=(
        pl.BlockSpec((RB, HH), lambda i: (i, 0)),
        pl.BlockSpec((RB, HH), lambda i: (i, 0)),
        pl.BlockSpec((RB, H), lambda i: (i, 0)),
    ),
    out_shape=(
        jax.ShapeDtypeStruct((NP, HH), f32),
        jax.ShapeDtypeStruct((NP, HH), f32),
        jax.ShapeDtypeStruct((NP, H), f32),
    ),
)


def _final_body(aa_ref, ab_ref, hrin_ref, b_ref, w0, b0, w1, b1, w2, b2,
                out_ref, pooled):
    i = pl.program_id(0)

    @pl.when(i == 0)
    def _():
        pooled[...] = jnp.full((G, H), -jnp.inf, f32)

    a = jnp.concatenate([aa_ref[...], ab_ref[...]], axis=1)
    h = jnp.maximum(a + hrin_ref[...], 0.0)
    bb = b_ref[...]
    gmin = jnp.min(bb)
    gmax = jnp.minimum(jnp.max(bb), G - 1)

    def upd(g, carry):
        m = bb == g
        v = jnp.max(jnp.where(m, h, -jnp.inf), axis=0, keepdims=True)
        pooled[pl.ds(g, 1), :] = jnp.maximum(pooled[pl.ds(g, 1), :], v)
        return carry

    lax.fori_loop(gmin, gmax + 1, upd, 0)

    @pl.when(i == NBLK - 1)
    def _():
        p = pooled[...]
        p1 = jnp.maximum(_dot(p, w0[...]) + b0[...], 0.0)
        p2 = jnp.maximum(_dot(p1, w1[...]) + b1[...], 0.0)
        out_ref[...] = _dot(p2, w2[...]) + b2[...]


_final = pl.pallas_call(
    _final_body,
    grid=(NBLK,),
    in_specs=[
        pl.BlockSpec((RB, HH), lambda i: (i, 0)),
        pl.BlockSpec((RB, HH), lambda i: (i, 0)),
        pl.BlockSpec((RB, H), lambda i: (i, 0)),
        pl.BlockSpec((RB, 1), lambda i: (i, 0)),
        pl.BlockSpec((H, H), lambda i: (0, 0)),
        pl.BlockSpec((1, H), lambda i: (0, 0)),
        pl.BlockSpec((H, H // 2), lambda i: (0, 0)),
        pl.BlockSpec((1, H // 2), lambda i: (0, 0)),
        pl.BlockSpec((H // 2, H), lambda i: (0, 0)),
        pl.BlockSpec((1, H), lambda i: (0, 0)),
    ],
    out_specs=pl.BlockSpec((G, H), lambda i: (0, 0)),
    out_shape=jax.ShapeDtypeStruct((G, H), f32),
    scratch_shapes=[pltpu.VMEM((G, H), f32)],
)


def kernel(x, dst_ports, tcp_flags, edge_index, batch, port_table, tcp_table,
           W_rel0, W_root0, b0, W_rel1, W_root1, b1, W_rel2, W_root2, b2,
           fc_W0, fc_b0, fc_W1, fc_b1, fc_W2, fc_b2):
    x_p = jnp.concatenate([x.astype(f32), jnp.zeros((NP - N, D), f32)], axis=0)
    ports_p = jnp.concatenate(
        [dst_ports.astype(i32), jnp.zeros((NP - N,), i32)], axis=0)
    flags_p = jnp.concatenate(
        [tcp_flags.astype(i32), jnp.zeros((NP - N,), i32)], axis=0
    ).reshape(NP, 1)
    src_p = jnp.concatenate(
        [edge_index[0].astype(i32), jnp.zeros((EP - E,), i32)], axis=0)
    dst_p = jnp.concatenate(
        [edge_index[1].astype(i32), jnp.full((EP - E,), NP - 1, i32)], axis=0)
    epk = jnp.stack([src_p, dst_p], axis=0).reshape(2, NW * ECH, 128)
    epk = epk.transpose(1, 0, 2)
    batch_p = jnp.concatenate(
        [batch.astype(i32), jnp.full((NP - N,), G, i32)], axis=0
    ).reshape(NP, 1)
    zeros = jnp.zeros((NP, HH), f32)

    Wxr0, Wpr0, Wfr0 = W_rel0[:D], W_rel0[D:D + PD], W_rel0[D + PD:]
    Wxt0, Wpt0, Wft0 = W_root0[:D], W_root0[D:D + PD], W_root0[D + PD:]
    TWr = tcp_table.astype(f32) @ Wfr0
    TWt = tcp_table.astype(f32) @ Wft0
    b0r = b0.reshape(1, H)
    b1r = b1.reshape(1, H)
    b2r = b2.reshape(1, H)
    fb0 = fc_b0.reshape(1, H)
    fb1 = fc_b1.reshape(1, H // 2)
    fW2p = jnp.concatenate([fc_W2, jnp.zeros((H // 2, H - NCLS), f32)], axis=1)
    fb2p = jnp.concatenate([fc_b2, jnp.zeros((H - NCLS,), f32)]).reshape(1, H)

    embed_sc = _get_embed_sc()
    edge_sc = _get_edge_sc()
    pe = embed_sc(ports_p, port_table.astype(f32))
    hwa, hwb, hr = _dense0(x_p, pe, flags_p, Wxr0, Wpr0, TWr, Wxt0, Wpt0,
                           TWt, b0r)
    aa, ab = edge_sc(hwa, hwb, epk, zeros)
    hwa, hwb, hr = _densemid(aa, ab, hr, W_rel1, W_root1, b1r)
    aa, ab = edge_sc(hwa, hwb, epk, zeros)
    hwa, hwb, hr = _densemid(aa, ab, hr, W_rel2, W_root2, b2r)
    aa, ab = edge_sc(hwa, hwb, epk, zeros)
    out = _final(aa, ab, hr, batch_p, fc_W0, fb0, fc_W1, fb1, fW2p, fb2p)
    return out[:, :NCLS]


# speedup vs baseline: 2.8265x; 1.1990x over previous
import functools

import jax
import jax.numpy as jnp
from jax import lax
from jax.experimental import pallas as pl
from jax.experimental.pallas import tpu as pltpu
from jax.experimental.pallas import tpu_sc as plsc

N = 10000
NP = 10240
D = 128
PD = 16
H = 128
E = 320000
G = 64
NCLS = 10

NC = 2
NS = 16
NW = NC * NS
EPW = 10240
EP = EPW * NW
ECH = EPW // 128
NI = 4
NBUF = 2
HH = H // 2
NCH = ECH * NC
NGRP = NCH // NI
RPS = NP // NS
RPW = NP // NW
RB = 512
NBLK = NP // RB

f32 = jnp.float32
i32 = jnp.int32

def _sc_mesh():
    return plsc.VectorSubcoreMesh(
        core_axis_name="c", subcore_axis_name="s", num_cores=NC, num_subcores=NS)


@functools.cache
def _get_embed_sc():
    @functools.partial(
        pl.kernel,
        out_type=jax.ShapeDtypeStruct((NP, PD), f32),
        scratch_types=[
            pltpu.VMEM((80,), i32),
            pltpu.VMEM((80, PD), f32),
            pltpu.SemaphoreType.DMA,
        ],
        mesh=_sc_mesh(),
        compiler_params=pltpu.CompilerParams(use_tc_tiling_on_sc=False),
    )
    def _embed_sc(ports_hbm, ptab_hbm, pe_hbm, pidx, perows, sem):
        c = lax.axis_index("c")
        s = lax.axis_index("s")
        wid = s * NC + c
        base = wid * RPW

        @pl.loop(0, RPW // 80)
        def _chunk(j):
            off = base + j * 80
            pltpu.sync_copy(ports_hbm.at[pl.ds(off, 80)], pidx)
            pltpu.async_copy(ptab_hbm.at[pidx], perows, sem).wait()
            pltpu.sync_copy(perows, pe_hbm.at[pl.ds(off, 80)])

    return _embed_sc


@functools.cache
def _get_edge_sc():
    @functools.partial(
        pl.kernel,
        out_type=(
            jax.ShapeDtypeStruct((NP, HH), f32),
            jax.ShapeDtypeStruct((NP, HH), f32),
        ),
        scratch_types=[
            pltpu.VMEM((NI, 128), i32),
            pltpu.VMEM((NI, 128), i32),
            pltpu.VMEM((NBUF, 128, HH), f32),
            pltpu.VMEM_SHARED((NP, HH), f32),
            pltpu.VMEM_SHARED((NP, HH), f32),
            pltpu.SemaphoreType.DMA((NI,)),
            pltpu.SemaphoreType.DMA((NBUF,)),
            pltpu.SemaphoreType.DMA((NBUF,)),
        ],
        mesh=_sc_mesh(),
        compiler_params=pltpu.CompilerParams(use_tc_tiling_on_sc=False),
    )
    def _edge_sc(hwa_hbm, hwb_hbm, ei_hbm, zeros_hbm, aa_hbm, ab_hbm,
                 sidx, didx, rows, hws, acc, isem, gsem, ssem):
        c = lax.axis_index("c")
        s = lax.axis_index("s")
        pltpu.sync_copy(zeros_hbm.at[pl.ds(s * RPS, RPS)],
                        acc.at[pl.ds(s * RPS, RPS)])

        @pl.when(c == 0)
        def _():
            pltpu.sync_copy(hwa_hbm.at[pl.ds(s * RPS, RPS)],
                            hws.at[pl.ds(s * RPS, RPS)])

        @pl.when(c == 1)
        def _():
            pltpu.sync_copy(hwb_hbm.at[pl.ds(s * RPS, RPS)],
                            hws.at[pl.ds(s * RPS, RPS)])

        plsc.subcore_barrier()
        qb = s * NCH

        def idxs_d(k, j):
            return pltpu.make_async_copy(
                ei_hbm.at[pl.ds((qb + k) * 128, 128)], sidx.at[j], isem.at[j])

        def idxd_d(k, j):
            return pltpu.make_async_copy(
                ei_hbm.at[pl.ds(EP + (qb + k) * 128, 128)], didx.at[j],
                isem.at[j])

        def idx_start(k, j):
            idxs_d(k, j).start()
            idxd_d(k, j).start()

        def idx_wait(k, j):
            idxs_d(k, j).wait()
            idxd_d(k, j).wait()

        def gat_d(b, j):
            return pltpu.make_async_copy(hws.at[sidx.at[j]], rows.at[b],
                                         gsem.at[b])

        def sct_d(b, j):
            return pltpu.make_async_copy(rows.at[b], acc.at[didx.at[j]],
                                         ssem.at[b])

        for j in range(NBUF):
            idx_start(j, j)

        @pl.loop(0, NGRP)
        def _grp(g):
            for u in range(NI):
                b = u % NBUF
                k = g * NI + u
                if u < NBUF:
                    @pl.when(g > 0)
                    def _():
                        sct_d(b, u + NBUF).wait()
                else:
                    sct_d(b, u - NBUF).wait()
                idx_wait(k, u)
                gat_d(b, u).start()
                j2 = (u + NBUF) % NI
                if u < NBUF:
                    idx_start(k + NBUF, j2)
                else:
                    @pl.when(g < NGRP - 1)
                    def _():
                        idx_start(k + NBUF, j2)
                if u >= 1:
                    bp = (u - 1) % NBUF
                    gat_d(bp, u - 1).wait()
                    sct_d(bp, u - 1).start(add=True)
                else:
                    @pl.when(g > 0)
                    def _():
                        gat_d((NI - 1) % NBUF, NI - 1).wait()
                        sct_d((NI - 1) % NBUF, NI - 1).start(add=True)

        gat_d((NI - 1) % NBUF, NI - 1).wait()
        sct_d((NI - 1) % NBUF, NI - 1).start(add=True)
        for u in range(NI - NBUF, NI):
            sct_d(u % NBUF, u).wait()

        plsc.subcore_barrier()

        @pl.when(c == 0)
        def _():
            pltpu.sync_copy(acc.at[pl.ds(s * RPS, RPS)],
                            aa_hbm.at[pl.ds(s * RPS, RPS)])

        @pl.when(c == 1)
        def _():
            pltpu.sync_copy(acc.at[pl.ds(s * RPS, RPS)],
                            ab_hbm.at[pl.ds(s * RPS, RPS)])

    return _edge_sc


def _dot(a, b):
    return jnp.dot(a, b, preferred_element_type=f32)


def _dense0_body(x_ref, pe_ref, fl_ref, wxr, wpr, twr, wxt, wpt, twt, br,
                 hwa_ref, hwb_ref, hr_ref):
    x = x_ref[...]
    pe = pe_ref[...]
    oh = (fl_ref[...] == lax.broadcasted_iota(i32, (1, 256), 1)).astype(f32)
    hw = _dot(x, wxr[...]) + _dot(pe, wpr[...]) + _dot(oh, twr[...])
    hwa_ref[...] = hw[:, :HH]
    hwb_ref[...] = hw[:, HH:]
    hr_ref[...] = (_dot(x, wxt[...]) + _dot(pe, wpt[...]) + _dot(oh, twt[...])
                   + br[...])


_dense0 = pl.pallas_call(
    _dense0_body,
    grid=(NBLK,),
    in_specs=[
        pl.BlockSpec((RB, D), lambda i: (i, 0)),
        pl.BlockSpec((RB, PD), lambda i: (i, 0)),
        pl.BlockSpec((RB, 1), lambda i: (i, 0)),
        pl.BlockSpec((D, H), lambda i: (0, 0)),
        pl.BlockSpec((PD, H), lambda i: (0, 0)),
        pl.BlockSpec((256, H), lambda i: (0, 0)),
        pl.BlockSpec((D, H), lambda i: (0, 0)),
        pl.BlockSpec((PD, H), lambda i: (0, 0)),
        pl.BlockSpec((256, H), lambda i: (0, 0)),
        pl.BlockSpec((1, H), lambda i: (0, 0)),
    ],
    out_specs=(
        pl.BlockSpec((RB, HH), lambda i: (i, 0)),
        pl.BlockSpec((RB, HH), lambda i: (i, 0)),
        pl.BlockSpec((RB, H), lambda i: (i, 0)),
    ),
    out_shape=(
        jax.ShapeDtypeStruct((NP, HH), f32),
        jax.ShapeDtypeStruct((NP, HH), f32),
        jax.ShapeDtypeStruct((NP, H), f32),
    ),
)


def _densemid_body(aa_ref, ab_ref, hrin_ref, wr, wt, b,
                   hwa_ref, hwb_ref, hr_ref):
    a = jnp.concatenate([aa_ref[...], ab_ref[...]], axis=1)
    h = jnp.maximum(a + hrin_ref[...], 0.0)
    hw = _dot(h, wr[...])
    hwa_ref[...] = hw[:, :HH]
    hwb_ref[...] = hw[:, HH:]
    hr_ref[...] = _dot(h, wt[...]) + b[...]


_densemid = pl.pallas_call(
    _densemid_body,
    grid=(NBLK,),
    in_specs=[
        pl.BlockSpec((RB, HH), lambda i: (i, 0)),
        pl.BlockSpec((RB, HH), lambda i: (i, 0)),
        pl.BlockSpec((RB, H), lambda i: (i, 0)),
        pl.BlockSpec((H, H), lambda i: (0, 0)),
        pl.BlockSpec((H, H), lambda i: (0, 0)),
        pl.BlockSpec((1, H), lambda i: (0, 0)),
    ],
    out_specs=(
        pl.BlockSpec((RB, HH), lambda i: (i, 0)),
        pl.BlockSpec((RB, HH), lambda i: (i, 0)),
        pl.BlockSpec((RB, H), lambda i: (i, 0)),
    ),
    out_shape=(
        jax.ShapeDtypeStruct((NP, HH), f32),
        jax.ShapeDtypeStruct((NP, HH), f32),
        jax.ShapeDtypeStruct((NP, H), f32),
    ),
)


def _final_body(aa_ref, ab_ref, hrin_ref, b_ref, w0, b0, w1, b1, w2, b2,
                out_ref, pooled):
    i = pl.program_id(0)

    @pl.when(i == 0)
    def _():
        pooled[...] = jnp.full((G, H), -jnp.inf, f32)

    a = jnp.concatenate([aa_ref[...], ab_ref[...]], axis=1)
    h = jnp.maximum(a + hrin_ref[...], 0.0)
    bb = b_ref[...]
    gmin = jnp.min(bb)
    gmax = jnp.minimum(jnp.max(bb), G - 1)

    def upd(g, carry):
        m = bb == g
        v = jnp.max(jnp.where(m, h, -jnp.inf), axis=0, keepdims=True)
        pooled[pl.ds(g, 1), :] = jnp.maximum(pooled[pl.ds(g, 1), :], v)
        return carry

    lax.fori_loop(gmin, gmax + 1, upd, 0)

    @pl.when(i == NBLK - 1)
    def _():
        p = pooled[...]
        p1 = jnp.maximum(_dot(p, w0[...]) + b0[...], 0.0)
        p2 = jnp.maximum(_dot(p1, w1[...]) + b1[...], 0.0)
        out_ref[...] = _dot(p2, w2[...]) + b2[...]


_final = pl.pallas_call(
    _final_body,
    grid=(NBLK,),
    in_specs=[
        pl.BlockSpec((RB, HH), lambda i: (i, 0)),
        pl.BlockSpec((RB, HH), lambda i: (i, 0)),
        pl.BlockSpec((RB, H), lambda i: (i, 0)),
        pl.BlockSpec((RB, 1), lambda i: (i, 0)),
        pl.BlockSpec((H, H), lambda i: (0, 0)),
        pl.BlockSpec((1, H), lambda i: (0, 0)),
        pl.BlockSpec((H, H // 2), lambda i: (0, 0)),
        pl.BlockSpec((1, H // 2), lambda i: (0, 0)),
        pl.BlockSpec((H // 2, H), lambda i: (0, 0)),
        pl.BlockSpec((1, H), lambda i: (0, 0)),
    ],
    out_specs=pl.BlockSpec((G, H), lambda i: (0, 0)),
    out_shape=jax.ShapeDtypeStruct((G, H), f32),
    scratch_shapes=[pltpu.VMEM((G, H), f32)],
)


def kernel(x, dst_ports, tcp_flags, edge_index, batch, port_table, tcp_table,
           W_rel0, W_root0, b0, W_rel1, W_root1, b1, W_rel2, W_root2, b2,
           fc_W0, fc_b0, fc_W1, fc_b1, fc_W2, fc_b2):
    x_p = jnp.concatenate([x.astype(f32), jnp.zeros((NP - N, D), f32)], axis=0)
    ports_p = jnp.concatenate(
        [dst_ports.astype(i32), jnp.zeros((NP - N,), i32)], axis=0)
    flags_p = jnp.concatenate(
        [tcp_flags.astype(i32), jnp.zeros((NP - N,), i32)], axis=0
    ).reshape(NP, 1)
    src_p = jnp.concatenate(
        [edge_index[0].astype(i32), jnp.zeros((EP - E,), i32)], axis=0)
    dst_p = jnp.concatenate(
        [edge_index[1].astype(i32), jnp.full((EP - E,), NP - 1, i32)], axis=0)
    ei = jnp.concatenate([src_p, dst_p])
    batch_p = jnp.concatenate(
        [batch.astype(i32), jnp.full((NP - N,), G, i32)], axis=0
    ).reshape(NP, 1)
    zeros = jnp.zeros((NP, HH), f32)

    Wxr0, Wpr0, Wfr0 = W_rel0[:D], W_rel0[D:D + PD], W_rel0[D + PD:]
    Wxt0, Wpt0, Wft0 = W_root0[:D], W_root0[D:D + PD], W_root0[D + PD:]
    TWr = tcp_table.astype(f32) @ Wfr0
    TWt = tcp_table.astype(f32) @ Wft0
    b0r = b0.reshape(1, H)
    b1r = b1.reshape(1, H)
    b2r = b2.reshape(1, H)
    fb0 = fc_b0.reshape(1, H)
    fb1 = fc_b1.reshape(1, H // 2)
    fW2p = jnp.concatenate([fc_W2, jnp.zeros((H // 2, H - NCLS), f32)], axis=1)
    fb2p = jnp.concatenate([fc_b2, jnp.zeros((H - NCLS,), f32)]).reshape(1, H)

    embed_sc = _get_embed_sc()
    edge_sc = _get_edge_sc()
    pe = embed_sc(ports_p, port_table.astype(f32))
    hwa, hwb, hr = _dense0(x_p, pe, flags_p, Wxr0, Wpr0, TWr, Wxt0, Wpt0,
                           TWt, b0r)
    aa, ab = edge_sc(hwa, hwb, ei, zeros)
    hwa, hwb, hr = _densemid(aa, ab, hr, W_rel1, W_root1, b1r)
    aa, ab = edge_sc(hwa, hwb, ei, zeros)
    hwa, hwb, hr = _densemid(aa, ab, hr, W_rel2, W_root2, b2r)
    aa, ab = edge_sc(hwa, hwb, ei, zeros)
    out = _final(aa, ab, hr, batch_p, fc_W0, fb0, fc_W1, fb1, fW2p, fb2p)
    return out[:, :NCLS]


# speedup vs baseline: 2.9243x; 1.0346x over previous
import functools

import jax
import jax.numpy as jnp
from jax import lax
from jax.experimental import pallas as pl
from jax.experimental.pallas import tpu as pltpu
from jax.experimental.pallas import tpu_sc as plsc

N = 10000
NP = 10240
D = 128
PD = 16
H = 128
E = 320000
G = 64
NCLS = 10

NC = 2
NS = 16
NW = NC * NS
EPW = 10240
EP = EPW * NW
ECH = EPW // 128
NI = 4
NBUF = 2
HH = H // 2
NCH = ECH * NC
NGRP = NCH // NI
RPS = NP // NS
RPW = NP // NW
RB = 1024
NBLK = NP // RB

f32 = jnp.float32
i32 = jnp.int32

def _sc_mesh():
    return plsc.VectorSubcoreMesh(
        core_axis_name="c", subcore_axis_name="s", num_cores=NC, num_subcores=NS)


@functools.cache
def _get_embed_sc():
    @functools.partial(
        pl.kernel,
        out_type=jax.ShapeDtypeStruct((NP, PD), f32),
        scratch_types=[
            pltpu.VMEM((80,), i32),
            pltpu.VMEM((80, PD), f32),
            pltpu.SemaphoreType.DMA,
        ],
        mesh=_sc_mesh(),
        compiler_params=pltpu.CompilerParams(use_tc_tiling_on_sc=False),
    )
    def _embed_sc(ports_hbm, ptab_hbm, pe_hbm, pidx, perows, sem):
        c = lax.axis_index("c")
        s = lax.axis_index("s")
        wid = s * NC + c
        base = wid * RPW

        @pl.loop(0, RPW // 80)
        def _chunk(j):
            off = base + j * 80
            pltpu.sync_copy(ports_hbm.at[pl.ds(off, 80)], pidx)
            pltpu.async_copy(ptab_hbm.at[pidx], perows, sem).wait()
            pltpu.sync_copy(perows, pe_hbm.at[pl.ds(off, 80)])

    return _embed_sc


@functools.cache
def _get_edge_sc():
    @functools.partial(
        pl.kernel,
        out_type=(
            jax.ShapeDtypeStruct((NP, HH), f32),
            jax.ShapeDtypeStruct((NP, HH), f32),
        ),
        scratch_types=[
            pltpu.VMEM((NI, 128), i32),
            pltpu.VMEM((NI, 128), i32),
            pltpu.VMEM((NBUF, 128, HH), f32),
            pltpu.VMEM_SHARED((NP, HH), f32),
            pltpu.VMEM_SHARED((NP, HH), f32),
            pltpu.SemaphoreType.DMA((NI,)),
            pltpu.SemaphoreType.DMA((NBUF,)),
            pltpu.SemaphoreType.DMA((NBUF,)),
        ],
        mesh=_sc_mesh(),
        compiler_params=pltpu.CompilerParams(use_tc_tiling_on_sc=False),
    )
    def _edge_sc(hwa_hbm, hwb_hbm, ei_hbm, zeros_hbm, aa_hbm, ab_hbm,
                 sidx, didx, rows, hws, acc, isem, gsem, ssem):
        c = lax.axis_index("c")
        s = lax.axis_index("s")
        pltpu.sync_copy(zeros_hbm.at[pl.ds(s * RPS, RPS)],
                        acc.at[pl.ds(s * RPS, RPS)])

        @pl.when(c == 0)
        def _():
            pltpu.sync_copy(hwa_hbm.at[pl.ds(s * RPS, RPS)],
                            hws.at[pl.ds(s * RPS, RPS)])

        @pl.when(c == 1)
        def _():
            pltpu.sync_copy(hwb_hbm.at[pl.ds(s * RPS, RPS)],
                            hws.at[pl.ds(s * RPS, RPS)])

        plsc.subcore_barrier()
        qb = s * NCH

        def idxs_d(k, j):
            return pltpu.make_async_copy(
                ei_hbm.at[pl.ds((qb + k) * 128, 128)], sidx.at[j], isem.at[j])

        def idxd_d(k, j):
            return pltpu.make_async_copy(
                ei_hbm.at[pl.ds(EP + (qb + k) * 128, 128)], didx.at[j],
                isem.at[j])

        def idx_start(k, j):
            idxs_d(k, j).start()
            idxd_d(k, j).start()

        def idx_wait(k, j):
            idxs_d(k, j).wait()
            idxd_d(k, j).wait()

        def gat_d(b, j):
            return pltpu.make_async_copy(hws.at[sidx.at[j]], rows.at[b],
                                         gsem.at[b])

        def sct_d(b, j):
            return pltpu.make_async_copy(rows.at[b], acc.at[didx.at[j]],
                                         ssem.at[b])

        for j in range(NBUF):
            idx_start(j, j)

        @pl.loop(0, NGRP)
        def _grp(g):
            for u in range(NI):
                b = u % NBUF
                k = g * NI + u
                if u < NBUF:
                    @pl.when(g > 0)
                    def _():
                        sct_d(b, u + NBUF).wait()
                else:
                    sct_d(b, u - NBUF).wait()
                idx_wait(k, u)
                gat_d(b, u).start()
                j2 = (u + NBUF) % NI
                if u < NBUF:
                    idx_start(k + NBUF, j2)
                else:
                    @pl.when(g < NGRP - 1)
                    def _():
                        idx_start(k + NBUF, j2)
                if u >= 1:
                    bp = (u - 1) % NBUF
                    gat_d(bp, u - 1).wait()
                    sct_d(bp, u - 1).start(add=True)
                else:
                    @pl.when(g > 0)
                    def _():
                        gat_d((NI - 1) % NBUF, NI - 1).wait()
                        sct_d((NI - 1) % NBUF, NI - 1).start(add=True)

        gat_d((NI - 1) % NBUF, NI - 1).wait()
        sct_d((NI - 1) % NBUF, NI - 1).start(add=True)
        for u in range(NI - NBUF, NI):
            sct_d(u % NBUF, u).wait()

        plsc.subcore_barrier()

        @pl.when(c == 0)
        def _():
            pltpu.sync_copy(acc.at[pl.ds(s * RPS, RPS)],
                            aa_hbm.at[pl.ds(s * RPS, RPS)])

        @pl.when(c == 1)
        def _():
            pltpu.sync_copy(acc.at[pl.ds(s * RPS, RPS)],
                            ab_hbm.at[pl.ds(s * RPS, RPS)])

    return _edge_sc


def _dot(a, b):
    return jnp.dot(a, b, preferred_element_type=f32)


def _dense0_body(x_ref, pe_ref, fl_ref, wxr, wpr, twr, wxt, wpt, twt, br,
                 hwa_ref, hwb_ref, hr_ref):
    x = x_ref[...]
    pe = pe_ref[...]
    oh = (fl_ref[...] == lax.broadcasted_iota(i32, (1, 256), 1)).astype(f32)
    hw = _dot(x, wxr[...]) + _dot(pe, wpr[...]) + _dot(oh, twr[...])
    hwa_ref[...] = hw[:, :HH]
    hwb_ref[...] = hw[:, HH:]
    hr_ref[...] = (_dot(x, wxt[...]) + _dot(pe, wpt[...]) + _dot(oh, twt[...])
                   + br[...])


_dense0 = pl.pallas_call(
    _dense0_body,
    grid=(NBLK,),
    in_specs=[
        pl.BlockSpec((RB, D), lambda i: (i, 0)),
        pl.BlockSpec((RB, PD), lambda i: (i, 0)),
        pl.BlockSpec((RB, 1), lambda i: (i, 0)),
        pl.BlockSpec((D, H), lambda i: (0, 0)),
        pl.BlockSpec((PD, H), lambda i: (0, 0)),
        pl.BlockSpec((256, H), lambda i: (0, 0)),
        pl.BlockSpec((D, H), lambda i: (0, 0)),
        pl.BlockSpec((PD, H), lambda i: (0, 0)),
        pl.BlockSpec((256, H), lambda i: (0, 0)),
        pl.BlockSpec((1, H), lambda i: (0, 0)),
    ],
    out_specs=(
        pl.BlockSpec((RB, HH), lambda i: (i, 0)),
        pl.BlockSpec((RB, HH), lambda i: (i, 0)),
        pl.BlockSpec((RB, H), lambda i: (i, 0)),
    ),
    out_shape=(
        jax.ShapeDtypeStruct((NP, HH), f32),
        jax.ShapeDtypeStruct((NP, HH), f32),
        jax.ShapeDtypeStruct((NP, H), f32),
    ),
)


def _densemid_body(aa_ref, ab_ref, hrin_ref, wr, wt, b,
                   hwa_ref, hwb_ref, hr_ref):
    a = jnp.concatenate([aa_ref[...], ab_ref[...]], axis=1)
    h = jnp.maximum(a + hrin_ref[...], 0.0)
    hw = _dot(h, wr[...])
    hwa_ref[...] = hw[:, :HH]
    hwb_ref[...] = hw[:, HH:]
    hr_ref[...] = _dot(h, wt[...]) + b[...]


_densemid = pl.pallas_call(
    _densemid_body,
    grid=(NBLK,),
    in_specs=[
        pl.BlockSpec((RB, HH), lambda i: (i, 0)),
        pl.BlockSpec((RB, HH), lambda i: (i, 0)),
        pl.BlockSpec((RB, H), lambda i: (i, 0)),
        pl.BlockSpec((H, H), lambda i: (0, 0)),
        pl.BlockSpec((H, H), lambda i: (0, 0)),
        pl.BlockSpec((1, H), lambda i: (0, 0)),
    ],
    out_specs=(
        pl.BlockSpec((RB, HH), lambda i: (i, 0)),
        pl.BlockSpec((RB, HH), lambda i: (i, 0)),
        pl.BlockSpec((RB, H), lambda i: (i, 0)),
    ),
    out_shape=(
        jax.ShapeDtypeStruct((NP, HH), f32),
        jax.ShapeDtypeStruct((NP, HH), f32),
        jax.ShapeDtypeStruct((NP, H), f32),
    ),
)


def _final_body(aa_ref, ab_ref, hrin_ref, b_ref, w0, b0, w1, b1, w2, b2,
                out_ref, pooled):
    i = pl.program_id(0)

    @pl.when(i == 0)
    def _():
        pooled[...] = jnp.full((G, H), -jnp.inf, f32)

    a = jnp.concatenate([aa_ref[...], ab_ref[...]], axis=1)
    h = jnp.maximum(a + hrin_ref[...], 0.0)
    bb = b_ref[...]
    gmin = jnp.min(bb)
    gmax = jnp.minimum(jnp.max(bb), G - 1)

    def upd(g, carry):
        m = bb == g
        v = jnp.max(jnp.where(m, h, -jnp.inf), axis=0, keepdims=True)
        pooled[pl.ds(g, 1), :] = jnp.maximum(pooled[pl.ds(g, 1), :], v)
        return carry

    lax.fori_loop(gmin, gmax + 1, upd, 0)

    @pl.when(i == NBLK - 1)
    def _():
        p = pooled[...]
        p1 = jnp.maximum(_dot(p, w0[...]) + b0[...], 0.0)
        p2 = jnp.maximum(_dot(p1, w1[...]) + b1[...], 0.0)
        out_ref[...] = _dot(p2, w2[...]) + b2[...]


_final = pl.pallas_call(
    _final_body,
    grid=(NBLK,),
    in_specs=[
        pl.BlockSpec((RB, HH), lambda i: (i, 0)),
        pl.BlockSpec((RB, HH), lambda i: (i, 0)),
        pl.BlockSpec((RB, H), lambda i: (i, 0)),
        pl.BlockSpec((RB, 1), lambda i: (i, 0)),
        pl.BlockSpec((H, H), lambda i: (0, 0)),
        pl.BlockSpec((1, H), lambda i: (0, 0)),
        pl.BlockSpec((H, H // 2), lambda i: (0, 0)),
        pl.BlockSpec((1, H // 2), lambda i: (0, 0)),
        pl.BlockSpec((H // 2, H), lambda i: (0, 0)),
        pl.BlockSpec((1, H), lambda i: (0, 0)),
    ],
    out_specs=pl.BlockSpec((G, H), lambda i: (0, 0)),
    out_shape=jax.ShapeDtypeStruct((G, H), f32),
    scratch_shapes=[pltpu.VMEM((G, H), f32)],
)


def kernel(x, dst_ports, tcp_flags, edge_index, batch, port_table, tcp_table,
           W_rel0, W_root0, b0, W_rel1, W_root1, b1, W_rel2, W_root2, b2,
           fc_W0, fc_b0, fc_W1, fc_b1, fc_W2, fc_b2):
    x_p = jnp.concatenate([x.astype(f32), jnp.zeros((NP - N, D), f32)], axis=0)
    ports_p = jnp.concatenate(
        [dst_ports.astype(i32), jnp.zeros((NP - N,), i32)], axis=0)
    flags_p = jnp.concatenate(
        [tcp_flags.astype(i32), jnp.zeros((NP - N,), i32)], axis=0
    ).reshape(NP, 1)
    src_p = jnp.concatenate(
        [edge_index[0].astype(i32), jnp.zeros((EP - E,), i32)], axis=0)
    dst_p = jnp.concatenate(
        [edge_index[1].astype(i32), jnp.full((EP - E,), NP - 1, i32)], axis=0)
    ei = jnp.concatenate([src_p, dst_p])
    batch_p = jnp.concatenate(
        [batch.astype(i32), jnp.full((NP - N,), G, i32)], axis=0
    ).reshape(NP, 1)
    zeros = jnp.zeros((NP, HH), f32)

    Wxr0, Wpr0, Wfr0 = W_rel0[:D], W_rel0[D:D + PD], W_rel0[D + PD:]
    Wxt0, Wpt0, Wft0 = W_root0[:D], W_root0[D:D + PD], W_root0[D + PD:]
    TWr = tcp_table.astype(f32) @ Wfr0
    TWt = tcp_table.astype(f32) @ Wft0
    b0r = b0.reshape(1, H)
    b1r = b1.reshape(1, H)
    b2r = b2.reshape(1, H)
    fb0 = fc_b0.reshape(1, H)
    fb1 = fc_b1.reshape(1, H // 2)
    fW2p = jnp.concatenate([fc_W2, jnp.zeros((H // 2, H - NCLS), f32)], axis=1)
    fb2p = jnp.concatenate([fc_b2, jnp.zeros((H - NCLS,), f32)]).reshape(1, H)

    embed_sc = _get_embed_sc()
    edge_sc = _get_edge_sc()
    pe = embed_sc(ports_p, port_table.astype(f32))
    hwa, hwb, hr = _dense0(x_p, pe, flags_p, Wxr0, Wpr0, TWr, Wxt0, Wpt0,
                           TWt, b0r)
    aa, ab = edge_sc(hwa, hwb, ei, zeros)
    hwa, hwb, hr = _densemid(aa, ab, hr, W_rel1, W_root1, b1r)
    aa, ab = edge_sc(hwa, hwb, ei, zeros)
    hwa, hwb, hr = _densemid(aa, ab, hr, W_rel2, W_root2, b2r)
    aa, ab = edge_sc(hwa, hwb, ei, zeros)
    out = _final(aa, ab, hr, batch_p, fc_W0, fb0, fc_W1, fb1, fW2p, fb2p)
    return out[:, :NCLS]
